# baseline (speedup 1.0000x reference)
import functools

import jax
import jax.numpy as jnp
from jax import lax
from jax.experimental import pallas as pl
from jax.experimental.pallas import tpu as pltpu
from jax.experimental.pallas import tpu_sc as plsc

_GAMMA = 12.0
_DIM = 64
_BATCH = 16384
_NC = 2
_NS = 16
_NW = _NC * _NS
_BPW = _BATCH // _NW
_NCHUNK = 4
_CHUNK = _BPW // _NCHUNK
_LANES = 16


def _body(head_hbm, ridx_hbm, ent_hbm, rel_hbm, out_hbm,
          hidx_v, ridx_v, hbuf, rbuf, tbuf, out_v, sem):
    wid = lax.axis_index("s") * _NC + lax.axis_index("c")
    pltpu.sync_copy(head_hbm.at[pl.ds(wid * _NCHUNK, _NCHUNK)], hidx_v)
    pltpu.sync_copy(ridx_hbm.at[pl.ds(wid * _NCHUNK, _NCHUNK)], ridx_v)

    copies = []
    for j in range(_NCHUNK):
        sl = pl.ds(j * _CHUNK, _CHUNK)
        copies.append(pltpu.async_copy(ent_hbm.at[hidx_v.at[j]], hbuf.at[sl], sem))
        copies.append(pltpu.async_copy(rel_hbm.at[ridx_v.at[j]], rbuf.at[sl], sem))
        copies.append(pltpu.async_copy(ent_hbm.at[ridx_v.at[j]], tbuf.at[sl], sem))
    for c in copies:
        c.wait()

    lanes = lax.broadcasted_iota(jnp.int32, (_LANES,), 0)

    def group(g, carry):
        rows = g * _LANES + lanes
        acc = jnp.zeros((_LANES,), jnp.float32)
        for d in range(_DIM):
            dvec = jnp.full((_LANES,), d, jnp.int32)
            h = plsc.load_gather(hbuf, [rows, dvec])
            r = plsc.load_gather(rbuf, [rows, dvec])
            t = plsc.load_gather(tbuf, [rows, dvec])
            acc = acc + jnp.abs(h + r - t)
        out_v[pl.ds(g * _LANES, _LANES)] = 1.0 / (1.0 + jnp.exp(acc - _GAMMA))
        return carry

    lax.fori_loop(0, _BPW // _LANES, group, 0)
    pltpu.sync_copy(out_v, out_hbm.at[pl.ds(wid * _BPW, _BPW)])


_transe_sc = functools.partial(
    pl.kernel,
    out_type=jax.ShapeDtypeStruct((_BATCH,), jnp.float32),
    mesh=plsc.VectorSubcoreMesh(core_axis_name="c", subcore_axis_name="s"),
    scratch_types=[
        pltpu.VMEM((_NCHUNK, _CHUNK), jnp.int32),
        pltpu.VMEM((_NCHUNK, _CHUNK), jnp.int32),
        pltpu.VMEM((_BPW, _DIM), jnp.float32),
        pltpu.VMEM((_BPW, _DIM), jnp.float32),
        pltpu.VMEM((_BPW, _DIM), jnp.float32),
        pltpu.VMEM((_BPW,), jnp.float32),
        pltpu.SemaphoreType.DMA,
    ],
    compiler_params=pltpu.CompilerParams(
        needs_layout_passes=False, use_tc_tiling_on_sc=False),
)(_body)


def kernel(x, emb_ent_real, emb_rel_real):
    head = x[:, 0].astype(jnp.int32).reshape(_NW * _NCHUNK, _CHUNK)
    ridx = x[:, 1].astype(jnp.int32).reshape(_NW * _NCHUNK, _CHUNK)
    return _transe_sc(head, ridx, emb_ent_real, emb_rel_real)
